# Initial kernel scaffold; baseline (speedup 1.0000x reference)
import jax
import jax.numpy as jnp
from jax.experimental import pallas as pl


def kernel(input, table):
    raise NotImplementedError("write your pallas kernel here")


# speedup vs baseline: 18.9781x; 18.9781x over previous
import functools

import jax
import jax.numpy as jnp
from jax import lax
from jax.experimental import pallas as pl
from jax.experimental.pallas import tpu as pltpu
from jax.experimental.pallas import tpu_sc as plsc

DIM = 32
C = 20
LANES = 16
NUM_CORES = 2
NUM_SUBCORES = 16
NUM_WORKERS = NUM_CORES * NUM_SUBCORES


def _tree_add(vals):
    vals = list(vals)
    while len(vals) > 1:
        nxt = [vals[i] + vals[i + 1] for i in range(0, len(vals) - 1, 2)]
        if len(vals) % 2:
            nxt.append(vals[-1])
        vals = nxt
    return vals[0]


@functools.partial(jax.jit, static_argnames=("n_tokens", "chunk"))
def _sc_embed(idx, table, *, n_tokens, chunk):
    tpw = n_tokens // NUM_WORKERS
    iters = tpw // chunk
    idxn = chunk * C

    mesh = plsc.VectorSubcoreMesh(
        core_axis_name="c", subcore_axis_name="s",
        num_cores=NUM_CORES, num_subcores=NUM_SUBCORES,
    )

    @functools.partial(
        pl.kernel,
        out_type=jax.ShapeDtypeStruct((n_tokens, DIM), jnp.float32),
        mesh=mesh,
        compiler_params=pltpu.CompilerParams(
            needs_layout_passes=False, use_tc_tiling_on_sc=False),
        scratch_types=[
            pltpu.VMEM((idxn,), jnp.int32),
            pltpu.VMEM((idxn, DIM), jnp.float32),
            pltpu.VMEM((chunk, DIM), jnp.float32),
            pltpu.VMEM((1, DIM), jnp.float32),
            pltpu.VMEM((chunk,), jnp.float32),
            pltpu.VMEM((chunk,), jnp.float32),
            pltpu.SemaphoreType.DMA,
        ],
    )
    def k(idx_hbm, table_hbm, out_hbm, idx_v, rows_v, out_v, t0_v,
          rden_v, n0f_v, gsem):
        wid = lax.axis_index("s") * NUM_CORES + lax.axis_index("c")
        pltpu.sync_copy(table_hbm.at[pl.ds(0, 1)], t0_v)
        t0a = t0_v[0, pl.ds(0, LANES)]
        t0b = t0_v[0, pl.ds(LANES, LANES)]
        lane = lax.iota(jnp.int32, 16)

        def do_chunk(it, _):
            tok_base = wid * tpw + it * chunk
            pltpu.sync_copy(idx_hbm.at[pl.ds(tok_base * C, idxn)], idx_v)
            pltpu.async_copy(table_hbm.at[idx_v], rows_v, gsem).wait()

            def grp_body(g, _):
                ibase = (g * LANES + lane) * C
                acc = jnp.zeros((LANES,), jnp.int32)
                for c in range(C):
                    v = plsc.load_gather(idx_v, [ibase + c])
                    acc = acc + jnp.where(v == 0, 1, 0)
                denom = jnp.maximum(C - acc, 1).astype(jnp.float32)
                rden_v[pl.ds(g * LANES, LANES)] = 1.0 / denom
                n0f_v[pl.ds(g * LANES, LANES)] = acc.astype(jnp.float32)
                return _

            lax.fori_loop(0, chunk // LANES, grp_body, None)

            def tok_body(t, _):
                rbase = t * C
                h0 = [rows_v[rbase + c, pl.ds(0, LANES)] for c in range(C)]
                h1 = [rows_v[rbase + c, pl.ds(LANES, LANES)] for c in range(C)]
                s0 = _tree_add(h0)
                s1 = _tree_add(h1)
                tfull = jnp.full((LANES,), t, jnp.int32)
                rden = plsc.load_gather(rden_v, [tfull])
                n0f = plsc.load_gather(n0f_v, [tfull])
                out_v[t, pl.ds(0, LANES)] = (s0 - n0f * t0a) * rden
                out_v[t, pl.ds(LANES, LANES)] = (s1 - n0f * t0b) * rden
                return _

            lax.fori_loop(0, chunk, tok_body, None)
            pltpu.sync_copy(out_v, out_hbm.at[pl.ds(tok_base, chunk)])
            return _

        lax.fori_loop(0, iters, do_chunk, None)

    return k(idx, table)


def kernel(input, table):
    B, T, Cdim = input.shape
    n_tokens = B * T
    idx = input.reshape(-1)
    out = _sc_embed(idx, table, n_tokens=n_tokens, chunk=64)
    return out.reshape(B, T, DIM)
